# baseline (speedup 1.0000x reference)
import functools

import jax
import jax.numpy as jnp
import numpy as np
from jax import lax
from jax.experimental import pallas as pl
from jax.experimental.pallas import tpu as pltpu
from jax.experimental.pallas import tpu_sc as plsc

B, L, D1, D2, DM = 4, 2048, 512, 1024, 2048
N = B * L
TBLK = 512
NBLK = N // TBLK + 1
M = N + TBLK
D2P = D2 // 2
DMP = DM // 2

_U1 = np.uint32(1)
_U16 = np.uint32(16)
_RNE = np.uint32(0x7FFF)
_HI = np.uint32(0xFFFF0000)


def _unpack_halves(words):
    w = lax.bitcast_convert_type(words, jnp.uint32)
    lo = lax.bitcast_convert_type(w << _U16, jnp.float32)
    hi = lax.bitcast_convert_type(w & _HI, jnp.float32)
    return lo, hi


def _pack_halves(lo, hi):
    bl = lax.bitcast_convert_type(lo, jnp.uint32)
    bh = lax.bitcast_convert_type(hi, jnp.uint32)
    bl = bl + _RNE + ((bl >> _U16) & _U1)
    bh = bh + _RNE + ((bh >> _U16) & _U1)
    return lax.bitcast_convert_type((bh & _HI) | (bl >> _U16), jnp.float32)


def _routing_body(tt_ref, dest_ref, bexp_ref):
    t = tt_ref[...]
    c = t
    k = 1
    while k < N:
        c = c + jnp.concatenate(
            [jnp.zeros((1, k), jnp.int32), c[:, : N - k]], axis=1)
        k *= 2
    n1 = jnp.sum(t)
    n0 = N - n1
    nblk0 = (n0 + TBLK - 1) // TBLK
    n0p = nblk0 * TBLK
    i = lax.broadcasted_iota(jnp.int32, (1, N), 1)
    dest_ref[...] = jnp.where(t == 0, i - c, n0p + c - 1)
    kk = lax.broadcasted_iota(jnp.int32, (1, 64), 1)
    bexp_ref[...] = (kk >= nblk0).astype(jnp.int32)


def _moe_body(bexp_ref, xs_ref, wa_ref, ba_ref, wb_ref, bb_ref, tt_ref, out_ref):
    del bexp_ref
    xb = xs_ref[...].astype(jnp.bfloat16)
    u = lax.dot_general(xb, wa_ref[0], (((1,), (0,)), ((), ())),
                        preferred_element_type=jnp.float32)
    u = u + ba_ref[0]
    u = 0.5 * u * (1.0 + lax.erf(u * 0.7071067811865476))
    h = lax.dot_general(u.astype(jnp.bfloat16), wb_ref[0],
                        (((1,), (0,)), ((), ())),
                        preferred_element_type=jnp.float32)
    h = h + bb_ref[0] + tt_ref[0]
    out_ref[...] = _pack_halves(h[:, :DMP], h[:, DMP:])


TLN = 1024


def _ln_body(y_ref, pos_ref, g_ref, b_ref, out_ref):
    yl, yh = _unpack_halves(y_ref[...])
    vl = yl + pos_ref[:, :DMP]
    vh = yh + pos_ref[:, DMP:]
    mu = (jnp.sum(vl, axis=1, keepdims=True) +
          jnp.sum(vh, axis=1, keepdims=True)) * (1.0 / DM)
    dl = vl - mu
    dh = vh - mu
    var = (jnp.sum(dl * dl, axis=1, keepdims=True) +
           jnp.sum(dh * dh, axis=1, keepdims=True)) * (1.0 / DM)
    r = lax.rsqrt(var + 1e-5)
    ol = dl * r * g_ref[:, :DMP] + b_ref[:, :DMP]
    oh = dh * r * g_ref[:, DMP:] + b_ref[:, DMP:]
    out_ref[...] = jnp.concatenate([ol, oh], axis=1)


def _scatter_rows(x_pk, dest64):
    mesh = plsc.VectorSubcoreMesh(core_axis_name="c", subcore_axis_name="s")

    @functools.partial(
        pl.kernel, mesh=mesh,
        out_type=jax.ShapeDtypeStruct((M, D2), jnp.float32),
        scratch_types=[
            pltpu.VMEM((64,), jnp.int32),
            pltpu.VMEM((64, D2), jnp.float32),
            pltpu.SemaphoreType.DMA,
        ])
    def scat(x_hbm, d_hbm, xs_hbm, idx_v, rows_v, sem):
        wid = lax.axis_index("s") * 2 + lax.axis_index("c")
        for cch in range(4):
            r = wid * 4 + cch
            pltpu.sync_copy(d_hbm.at[r], idx_v)
            pltpu.sync_copy(x_hbm.at[pl.ds(r * 64, 64)], rows_v)
            pltpu.async_copy(rows_v, xs_hbm.at[idx_v], sem).wait()

    return scat(x_pk, dest64)


def _unpermute_rows(h_sorted, dest32):
    mesh = plsc.VectorSubcoreMesh(core_axis_name="c", subcore_axis_name="s")

    @functools.partial(
        pl.kernel, mesh=mesh,
        out_type=jax.ShapeDtypeStruct((N, DMP), jnp.float32),
        scratch_types=[
            pltpu.VMEM((8, 32), jnp.int32),
            pltpu.VMEM((32, DMP), jnp.float32),
            pltpu.VMEM((32, DMP), jnp.float32),
            pltpu.SemaphoreType.DMA,
            pltpu.SemaphoreType.DMA,
        ])
    def unp(h_hbm, d_hbm, o_hbm, idx_v, rows0, rows1, gsem, ssem):
        wid = lax.axis_index("s") * 2 + lax.axis_index("c")
        bufs = [rows0, rows1]
        pltpu.sync_copy(d_hbm.at[pl.ds(wid * 8, 8)], idx_v)
        nch = 8
        gh = [None] * nch
        sh = [None] * nch
        gh[0] = pltpu.async_copy(h_hbm.at[idx_v.at[0]], bufs[0], gsem)
        for cch in range(nch):
            gh[cch].wait()
            if cch < nch - 1:
                if cch >= 1:
                    sh[cch - 1].wait()
                gh[cch + 1] = pltpu.async_copy(
                    h_hbm.at[idx_v.at[cch + 1]], bufs[(cch + 1) % 2], gsem)
            sh[cch] = pltpu.async_copy(
                bufs[cch % 2], o_hbm.at[pl.ds((wid * 8 + cch) * 32, 32)], ssem)
        sh[nch - 2].wait()
        sh[nch - 1].wait()

    return unp(h_sorted, dest32)


def kernel(x, token_type_ids, W1a, b1a, W1b, b1b, W2a, b2a, W2b, b2b,
           type_table, pos_table, gamma, beta):
    x_flat = x.reshape(N, D2)
    tt = token_type_ids.reshape(1, N)

    dest, bexp = pl.pallas_call(
        _routing_body,
        out_shape=(jax.ShapeDtypeStruct((1, N), jnp.int32),
                   jax.ShapeDtypeStruct((1, 64), jnp.int32)),
    )(tt)
    dest64 = dest.reshape(128, 64)
    dest32 = dest.reshape(256, 32)
    bexp1 = bexp.reshape(64)[:NBLK]

    xs = _scatter_rows(x_flat, dest64)

    wa = jnp.stack([jnp.pad(W1a.astype(jnp.bfloat16), ((0, D2 - D1), (0, 0))),
                    W2a.astype(jnp.bfloat16)])
    wb = jnp.stack([W1b.astype(jnp.bfloat16), W2b.astype(jnp.bfloat16)])
    ba = jnp.stack([b1a, b2a]).reshape(2, 1, DM)
    bb = jnp.stack([b1b, b2b]).reshape(2, 1, DM)
    tt3 = type_table.reshape(2, 1, DM)

    grid_spec = pltpu.PrefetchScalarGridSpec(
        num_scalar_prefetch=1,
        grid=(NBLK,),
        in_specs=[
            pl.BlockSpec((TBLK, D2), lambda i, s: (i, 0)),
            pl.BlockSpec((1, D2, DM), lambda i, s: (s[i], 0, 0)),
            pl.BlockSpec((1, 1, DM), lambda i, s: (s[i], 0, 0)),
            pl.BlockSpec((1, DM, DM), lambda i, s: (s[i], 0, 0)),
            pl.BlockSpec((1, 1, DM), lambda i, s: (s[i], 0, 0)),
            pl.BlockSpec((1, 1, DM), lambda i, s: (s[i], 0, 0)),
        ],
        out_specs=pl.BlockSpec((TBLK, DMP), lambda i, s: (i, 0)),
    )
    h = pl.pallas_call(
        _moe_body, grid_spec=grid_spec,
        out_shape=jax.ShapeDtypeStruct((M, DMP), jnp.float32),
    )(bexp1, xs, wa, ba, wb, bb, tt3)

    y = _unpermute_rows(h, dest32)

    out = pl.pallas_call(
        _ln_body,
        grid=(L // TLN, B),
        in_specs=[
            pl.BlockSpec((TLN, DMP), lambda i, j: (j * (L // TLN) + i, 0)),
            pl.BlockSpec((TLN, DM), lambda i, j: (i, 0)),
            pl.BlockSpec((1, DM), lambda i, j: (0, 0)),
            pl.BlockSpec((1, DM), lambda i, j: (0, 0)),
        ],
        out_specs=pl.BlockSpec((TLN, DM), lambda i, j: (j * (L // TLN) + i, 0)),
        out_shape=jax.ShapeDtypeStruct((N, DM), jnp.float32),
    )(y, pos_table, gamma.reshape(1, DM), beta.reshape(1, DM))

    return out.reshape(B, L, DM)


# speedup vs baseline: 1.0060x; 1.0060x over previous
import functools

import jax
import jax.numpy as jnp
import numpy as np
from jax import lax
from jax.experimental import pallas as pl
from jax.experimental.pallas import tpu as pltpu
from jax.experimental.pallas import tpu_sc as plsc

B, L, D1, D2, DM = 4, 2048, 512, 1024, 2048
N = B * L
TBLK = 512
NBLK = N // TBLK + 1
M = N + TBLK
D2P = D2 // 2
DMP = DM // 2

_U1 = np.uint32(1)
_U16 = np.uint32(16)
_RNE = np.uint32(0x7FFF)
_HI = np.uint32(0xFFFF0000)


def _unpack_halves(words):
    w = lax.bitcast_convert_type(words, jnp.uint32)
    lo = lax.bitcast_convert_type(w << _U16, jnp.float32)
    hi = lax.bitcast_convert_type(w & _HI, jnp.float32)
    return lo, hi


def _pack_halves(lo, hi):
    bl = lax.bitcast_convert_type(lo, jnp.uint32)
    bh = lax.bitcast_convert_type(hi, jnp.uint32)
    bl = bl + _RNE + ((bl >> _U16) & _U1)
    bh = bh + _RNE + ((bh >> _U16) & _U1)
    return lax.bitcast_convert_type((bh & _HI) | (bl >> _U16), jnp.float32)


def _routing_body(tt_ref, dest_ref, bexp_ref):
    t = tt_ref[...]
    c = t
    k = 1
    while k < N:
        c = c + jnp.concatenate(
            [jnp.zeros((1, k), jnp.int32), c[:, : N - k]], axis=1)
        k *= 2
    n1 = jnp.sum(t)
    n0 = N - n1
    nblk0 = (n0 + TBLK - 1) // TBLK
    n0p = nblk0 * TBLK
    i = lax.broadcasted_iota(jnp.int32, (1, N), 1)
    dest_ref[...] = jnp.where(t == 0, i - c, n0p + c - 1)
    kk = lax.broadcasted_iota(jnp.int32, (1, 64), 1)
    bexp_ref[...] = (kk >= nblk0).astype(jnp.int32)


def _moe_body(bexp_ref, xs_ref, wa_ref, ba_ref, wb_ref, bb_ref, tt_ref, out_ref):
    del bexp_ref
    xb = xs_ref[...].astype(jnp.bfloat16)
    u = lax.dot_general(xb, wa_ref[0], (((1,), (0,)), ((), ())),
                        preferred_element_type=jnp.float32)
    u = u + ba_ref[0]
    u = 0.5 * u * (1.0 + lax.erf(u * 0.7071067811865476))
    h = lax.dot_general(u.astype(jnp.bfloat16), wb_ref[0],
                        (((1,), (0,)), ((), ())),
                        preferred_element_type=jnp.float32)
    h = h + bb_ref[0] + tt_ref[0]
    out_ref[...] = _pack_halves(h[:, :DMP], h[:, DMP:])


TLN = 1024


def _ln_body(y_ref, pos_ref, g_ref, b_ref, out_ref):
    yl, yh = _unpack_halves(y_ref[...])
    vl = yl + pos_ref[:, :DMP]
    vh = yh + pos_ref[:, DMP:]
    mu = (jnp.sum(vl, axis=1, keepdims=True) +
          jnp.sum(vh, axis=1, keepdims=True)) * (1.0 / DM)
    dl = vl - mu
    dh = vh - mu
    var = (jnp.sum(dl * dl, axis=1, keepdims=True) +
           jnp.sum(dh * dh, axis=1, keepdims=True)) * (1.0 / DM)
    r = lax.rsqrt(var + 1e-5)
    ol = dl * r * g_ref[:, :DMP] + b_ref[:, :DMP]
    oh = dh * r * g_ref[:, DMP:] + b_ref[:, DMP:]
    out_ref[...] = jnp.concatenate([ol, oh], axis=1)


def _scatter_rows(x_pk, dest64):
    mesh = plsc.VectorSubcoreMesh(core_axis_name="c", subcore_axis_name="s")

    @functools.partial(
        pl.kernel, mesh=mesh,
        out_type=jax.ShapeDtypeStruct((M, D2), jnp.float32),
        scratch_types=[
            pltpu.VMEM((64,), jnp.int32),
            pltpu.VMEM((64, D2), jnp.float32),
            pltpu.SemaphoreType.DMA,
        ])
    def scat(x_hbm, d_hbm, xs_hbm, idx_v, rows_v, sem):
        wid = lax.axis_index("s") * 2 + lax.axis_index("c")
        for cch in range(4):
            r = wid * 4 + cch
            pltpu.sync_copy(d_hbm.at[r], idx_v)
            pltpu.sync_copy(x_hbm.at[pl.ds(r * 64, 64)], rows_v)
            pltpu.async_copy(rows_v, xs_hbm.at[idx_v], sem).wait()

    return scat(x_pk, dest64)


def _unpermute_rows(h_sorted, dest64):
    mesh = plsc.VectorSubcoreMesh(core_axis_name="c", subcore_axis_name="s")

    @functools.partial(
        pl.kernel, mesh=mesh,
        out_type=jax.ShapeDtypeStruct((N, DMP), jnp.float32),
        scratch_types=[
            pltpu.VMEM((64,), jnp.int32),
            pltpu.VMEM((64, DMP), jnp.float32),
            pltpu.SemaphoreType.DMA,
        ])
    def unp(h_hbm, d_hbm, o_hbm, idx_v, rows_v, sem):
        wid = lax.axis_index("s") * 2 + lax.axis_index("c")
        for cch in range(4):
            r = wid * 4 + cch
            pltpu.sync_copy(d_hbm.at[r], idx_v)
            pltpu.async_copy(h_hbm.at[idx_v], rows_v, sem).wait()
            pltpu.sync_copy(rows_v, o_hbm.at[pl.ds(r * 64, 64)])

    return unp(h_sorted, dest64)


def kernel(x, token_type_ids, W1a, b1a, W1b, b1b, W2a, b2a, W2b, b2b,
           type_table, pos_table, gamma, beta):
    x_flat = x.reshape(N, D2)
    tt = token_type_ids.reshape(1, N)

    dest, bexp = pl.pallas_call(
        _routing_body,
        out_shape=(jax.ShapeDtypeStruct((1, N), jnp.int32),
                   jax.ShapeDtypeStruct((1, 64), jnp.int32)),
    )(tt)
    dest64 = dest.reshape(128, 64)
    bexp1 = bexp.reshape(64)[:NBLK]

    xs = _scatter_rows(x_flat, dest64)

    wa = jnp.stack([jnp.pad(W1a.astype(jnp.bfloat16), ((0, D2 - D1), (0, 0))),
                    W2a.astype(jnp.bfloat16)])
    wb = jnp.stack([W1b.astype(jnp.bfloat16), W2b.astype(jnp.bfloat16)])
    ba = jnp.stack([b1a, b2a]).reshape(2, 1, DM)
    bb = jnp.stack([b1b, b2b]).reshape(2, 1, DM)
    tt3 = type_table.reshape(2, 1, DM)

    grid_spec = pltpu.PrefetchScalarGridSpec(
        num_scalar_prefetch=1,
        grid=(NBLK,),
        in_specs=[
            pl.BlockSpec((TBLK, D2), lambda i, s: (i, 0)),
            pl.BlockSpec((1, D2, DM), lambda i, s: (s[i], 0, 0)),
            pl.BlockSpec((1, 1, DM), lambda i, s: (s[i], 0, 0)),
            pl.BlockSpec((1, DM, DM), lambda i, s: (s[i], 0, 0)),
            pl.BlockSpec((1, 1, DM), lambda i, s: (s[i], 0, 0)),
            pl.BlockSpec((1, 1, DM), lambda i, s: (s[i], 0, 0)),
        ],
        out_specs=pl.BlockSpec((TBLK, DMP), lambda i, s: (i, 0)),
    )
    h = pl.pallas_call(
        _moe_body, grid_spec=grid_spec,
        out_shape=jax.ShapeDtypeStruct((M, DMP), jnp.float32),
    )(bexp1, xs, wa, ba, wb, bb, tt3)

    y = _unpermute_rows(h, dest64)

    out = pl.pallas_call(
        _ln_body,
        grid=(L // TLN, B),
        in_specs=[
            pl.BlockSpec((TLN, DMP), lambda i, j: (j * (L // TLN) + i, 0)),
            pl.BlockSpec((TLN, DM), lambda i, j: (i, 0)),
            pl.BlockSpec((1, DM), lambda i, j: (0, 0)),
            pl.BlockSpec((1, DM), lambda i, j: (0, 0)),
        ],
        out_specs=pl.BlockSpec((TLN, DM), lambda i, j: (j * (L // TLN) + i, 0)),
        out_shape=jax.ShapeDtypeStruct((N, DM), jnp.float32),
    )(y, pos_table, gamma.reshape(1, DM), beta.reshape(1, DM))

    return out.reshape(B, L, DM)
